# baseline (speedup 1.0000x reference)
import functools

import jax
import jax.numpy as jnp
from jax import lax
from jax.experimental import pallas as pl
from jax.experimental.pallas import tpu as pltpu
from jax.experimental.pallas import tpu_sc as plsc

N_NODES = 10000
D = 128
DH = 64
N_CORES = 2
N_SUBCORES = 16
NW = N_CORES * N_SUBCORES
N_PAD = 10240
RPT = N_PAD // N_SUBCORES
N_EDGES = 320000
CHUNK = 128
EPT = 20480
N_CHUNKS = EPT // CHUNK
E_PAD = N_SUBCORES * EPT
NBUF = 5
NIDX = 2 * NBUF
N_OUTER = N_CHUNKS // NBUF
N_SP = 3

_mesh = plsc.VectorSubcoreMesh(core_axis_name="c", subcore_axis_name="s")


@functools.partial(
    pl.kernel,
    out_type=jax.ShapeDtypeStruct((N_CORES * N_PAD, DH), jnp.float32),
    mesh=_mesh,
    scratch_types=[
        pltpu.VMEM((NIDX, CHUNK), jnp.int32),
        pltpu.VMEM((NIDX, CHUNK), jnp.int32),
        pltpu.VMEM((NBUF, CHUNK, DH), jnp.float32),
        pltpu.VMEM_SHARED((N_PAD, DH), jnp.float32),
        pltpu.VMEM_SHARED((N_PAD, DH), jnp.float32),
        pltpu.SemaphoreType.DMA((NIDX,)),
        pltpu.SemaphoreType.DMA((NBUF,)),
        pltpu.SemaphoreType.DMA((NBUF,)),
    ],
    compiler_params=pltpu.CompilerParams(use_tc_tiling_on_sc=False,
                                         needs_layout_passes=False),
)
def _sc_agg(f_hbm, src_hbm, srco_hbm, dst_hbm, out_hbm,
            src_v, dst_v, rows, feat, acc, isem, gsem, ssem):
    c = lax.axis_index("c")
    s = lax.axis_index("s")
    wid = c * N_SUBCORES + s

    pltpu.sync_copy(f_hbm.at[pl.ds(c * N_PAD + s * RPT, RPT)],
                    feat.at[pl.ds(s * RPT, RPT)])
    pltpu.sync_copy(f_hbm.at[pl.ds(c * N_PAD + s * RPT, RPT)],
                    acc.at[pl.ds(s * RPT, RPT)])
    plsc.subcore_barrier()

    def islot(i):
        if isinstance(i, int):
            return i % NIDX
        return lax.rem(i, NIDX)

    def idx_start(i, b):
        j = islot(i)
        if b < N_SP:
            pltpu.async_copy(src_hbm.at[s, i], src_v.at[j], isem.at[j])
        else:
            pltpu.async_copy(srco_hbm.at[wid, i], src_v.at[j], isem.at[j])
        pltpu.async_copy(dst_hbm.at[s, i], dst_v.at[j], isem.at[j])

    def idx_wait(i, b):
        j = islot(i)
        if b < N_SP:
            pltpu.make_async_copy(src_hbm.at[s, i], src_v.at[j],
                                  isem.at[j]).wait()
        else:
            pltpu.make_async_copy(srco_hbm.at[wid, i], src_v.at[j],
                                  isem.at[j]).wait()
        pltpu.make_async_copy(dst_hbm.at[s, i], dst_v.at[j],
                              isem.at[j]).wait()

    def gather_start(i, b):
        src = feat if b < N_SP else f_hbm
        pltpu.async_copy(src.at[src_v.at[islot(i)]], rows.at[b],
                         gsem.at[b])

    def gather_wait(i, b):
        src = feat if b < N_SP else f_hbm
        pltpu.make_async_copy(src.at[src_v.at[islot(i)]], rows.at[b],
                              gsem.at[b]).wait()

    def scatter_start(i, b):
        pltpu.async_copy(rows.at[b], acc.at[dst_v.at[islot(i)]],
                         ssem.at[b], add=True)

    def scatter_wait(i, b):
        pltpu.make_async_copy(rows.at[b], acc.at[dst_v.at[islot(i)]],
                              ssem.at[b]).wait()

    for j in range(NIDX):
        idx_start(j, j % NBUF)
    for b in range(NBUF):
        idx_wait(b, b)
        gather_start(b, b)

    def outer(g, carry):
        i0 = g * NBUF
        for b in range(NBUF):
            gather_wait(i0 + b, b)
            scatter_start(i0 + b, b)
        for b in range(NBUF):
            i2 = i0 + b + NBUF

            @pl.when(i2 < N_CHUNKS)
            def _():
                scatter_wait(i0 + b, b)
                idx_wait(i2, b)
                gather_start(i2, b)

                @pl.when(i2 + NBUF < N_CHUNKS)
                def _():
                    idx_start(i2 + NBUF, b)

        return carry

    lax.fori_loop(0, N_OUTER, outer, 0)
    for b in range(NBUF):
        scatter_wait(N_CHUNKS - NBUF + b, b)
    plsc.subcore_barrier()

    pltpu.sync_copy(acc.at[pl.ds(s * RPT, RPT)],
                    out_hbm.at[pl.ds(c * N_PAD + s * RPT, RPT)])


def _dense_body(agg_ref, w_ref, b_ref, o_ref, *, relu, split_out):
    h = jnp.concatenate([agg_ref[0:N_PAD, :], agg_ref[N_PAD:, :]], axis=1)
    o = jnp.dot(h, w_ref[...], preferred_element_type=jnp.float32) + b_ref[...]
    if relu:
        o = jnp.maximum(o, 0.0)
    if split_out:
        o_ref[0:N_PAD, :] = o[:, :DH]
        o_ref[N_PAD:, :] = o[:, DH:]
    else:
        o_ref[...] = o


def _dense(agg, w, b, relu, split_out):
    dout = w.shape[1]
    out_rows = N_CORES * N_PAD if split_out else N_PAD
    out_cols = DH if split_out else dout
    return pl.pallas_call(
        functools.partial(_dense_body, relu=relu, split_out=split_out),
        out_shape=jax.ShapeDtypeStruct((out_rows, out_cols), jnp.float32),
    )(agg, w, b.reshape(1, dout))


def kernel(x, edge_index, W1, b1, W2, b2, W3, b3, W4, b4):
    src = edge_index[0].astype(jnp.int32)
    dst = edge_index[1].astype(jnp.int32)
    pad = E_PAD - N_EDGES
    src_p = jnp.concatenate([src, jnp.zeros((pad,), jnp.int32)])
    dst_p = jnp.concatenate([dst, jnp.full((pad,), N_NODES, jnp.int32)])
    src2 = src_p.reshape(N_SUBCORES, N_CHUNKS, CHUNK)
    srco = jnp.stack([src_p, src_p + N_PAD]).reshape(NW, N_CHUNKS, CHUNK)
    dst2 = dst_p.reshape(N_SUBCORES, N_CHUNKS, CHUNK)
    x_pad = jnp.concatenate(
        [x, jnp.zeros((N_PAD - N_NODES, D), jnp.float32)], axis=0)
    f = jnp.concatenate([x_pad[:, :DH], x_pad[:, DH:]], axis=0)
    for w, b, relu, split in ((W1, b1, True, True), (W2, b2, True, True),
                              (W3, b3, True, True), (W4, b4, False, False)):
        agg = _sc_agg(f, src2, srco, dst2)
        f = _dense(agg, w, b, relu, split)
    return f[:N_NODES]


# speedup vs baseline: 1.1328x; 1.1328x over previous
import functools

import jax
import jax.numpy as jnp
import numpy as np
from jax import lax
from jax.experimental import pallas as pl
from jax.experimental.pallas import tpu as pltpu
from jax.experimental.pallas import tpu_sc as plsc

N_NODES = 10000
D = 128
DH = 64
N_CORES = 2
N_SUBCORES = 16
NW = N_CORES * N_SUBCORES
N_PAD = 10240
RPT = N_PAD // N_SUBCORES
N_EDGES = 320000
CHUNK = 128
EPT = 20480
N_CHUNKS = EPT // CHUNK
E_PAD = N_SUBCORES * EPT
NBUF = 5
NIDX = 2 * NBUF
N_OUTER = N_CHUNKS // NBUF

_PERM = np.empty((DH,), dtype=np.int32)
for _h in range(2):
    for _k in range(16):
        _PERM[_h * 32 + 2 * _k] = _h * 32 + _k
        _PERM[_h * 32 + 2 * _k + 1] = _h * 32 + 16 + _k

_mesh = plsc.VectorSubcoreMesh(core_axis_name="c", subcore_axis_name="s")


@functools.partial(
    pl.kernel,
    out_type=jax.ShapeDtypeStruct((N_CORES * N_PAD, DH), jnp.float32),
    mesh=_mesh,
    scratch_types=[
        pltpu.VMEM((NIDX, CHUNK), jnp.int32),
        pltpu.VMEM((NIDX, CHUNK), jnp.int32),
        pltpu.VMEM((NBUF, CHUNK, DH), jnp.bfloat16),
        pltpu.VMEM((NBUF, CHUNK, DH), jnp.float32),
        pltpu.VMEM_SHARED((N_PAD, DH), jnp.bfloat16),
        pltpu.VMEM_SHARED((N_PAD, DH), jnp.float32),
        pltpu.SemaphoreType.DMA((NIDX,)),
        pltpu.SemaphoreType.DMA((NBUF,)),
        pltpu.SemaphoreType.DMA((NBUF,)),
    ],
    compiler_params=pltpu.CompilerParams(use_tc_tiling_on_sc=False,
                                         needs_layout_passes=False),
)
def _sc_agg(f_hbm, fb_hbm, src_hbm, dst_hbm, out_hbm,
            src_v, dst_v, rows16, rows32, feat, acc, isem, gsem, ssem):
    c = lax.axis_index("c")
    s = lax.axis_index("s")

    pltpu.sync_copy(fb_hbm.at[pl.ds(c * N_PAD + s * RPT, RPT)],
                    feat.at[pl.ds(s * RPT, RPT)])
    pltpu.sync_copy(f_hbm.at[pl.ds(c * N_PAD + s * RPT, RPT)],
                    acc.at[pl.ds(s * RPT, RPT)])
    plsc.subcore_barrier()

    def islot(i):
        if isinstance(i, int):
            return i % NIDX
        return lax.rem(i, NIDX)

    def idx_start(i):
        j = islot(i)
        pltpu.async_copy(src_hbm.at[s, i], src_v.at[j], isem.at[j])
        pltpu.async_copy(dst_hbm.at[s, i], dst_v.at[j], isem.at[j])

    def idx_wait(i):
        j = islot(i)
        pltpu.make_async_copy(src_hbm.at[s, i], src_v.at[j],
                              isem.at[j]).wait()
        pltpu.make_async_copy(dst_hbm.at[s, i], dst_v.at[j],
                              isem.at[j]).wait()

    def gather_start(i, b):
        pltpu.async_copy(feat.at[src_v.at[islot(i)]], rows16.at[b],
                         gsem.at[b])

    def gather_wait(i, b):
        pltpu.make_async_copy(feat.at[src_v.at[islot(i)]], rows16.at[b],
                              gsem.at[b]).wait()

    def scatter_start(i, b):
        pltpu.async_copy(rows32.at[b], acc.at[dst_v.at[islot(i)]],
                         ssem.at[b], add=True)

    def scatter_wait(i, b):
        pltpu.make_async_copy(rows32.at[b], acc.at[dst_v.at[islot(i)]],
                              ssem.at[b]).wait()

    def widen(b):
        @plsc.parallel_loop(0, CHUNK // 4, 1, unroll=4)
        def _(r):
            for rr in range(4):
                row = r * 4 + rr
                for h in range(2):
                    v16 = rows16[b, row, pl.ds(h * 32, 32)]
                    w = plsc.bitcast(v16, jnp.int32)
                    even = plsc.bitcast(w << 16, jnp.float32)
                    odd = plsc.bitcast(w & jnp.int32(-65536), jnp.float32)
                    rows32[b, row, pl.ds(h * 32, 16)] = even
                    rows32[b, row, pl.ds(h * 32 + 16, 16)] = odd

    for j in range(NIDX):
        idx_start(j)
    for b in range(NBUF):
        idx_wait(b)
        gather_start(b, b)

    def outer(g, carry):
        i0 = g * NBUF
        for b in range(NBUF):
            gather_wait(i0 + b, b)
            widen(b)
            scatter_start(i0 + b, b)
        for b in range(NBUF):
            i2 = i0 + b + NBUF

            @pl.when(i2 < N_CHUNKS)
            def _():
                scatter_wait(i0 + b, b)
                idx_wait(i2)
                gather_start(i2, b)

                @pl.when(i2 + NBUF < N_CHUNKS)
                def _():
                    idx_start(i2 + NBUF)

        return carry

    lax.fori_loop(0, N_OUTER, outer, 0)
    for b in range(NBUF):
        scatter_wait(N_CHUNKS - NBUF + b, b)
    plsc.subcore_barrier()

    pltpu.sync_copy(acc.at[pl.ds(s * RPT, RPT)],
                    out_hbm.at[pl.ds(c * N_PAD + s * RPT, RPT)])


def _dense_body(agg_ref, w_ref, b_ref, o_ref, *, relu, split_out):
    h = jnp.concatenate([agg_ref[0:N_PAD, :], agg_ref[N_PAD:, :]], axis=1)
    o = jnp.dot(h, w_ref[...], preferred_element_type=jnp.float32) + b_ref[...]
    if relu:
        o = jnp.maximum(o, 0.0)
    if split_out:
        o_ref[0:N_PAD, :] = o[:, :DH]
        o_ref[N_PAD:, :] = o[:, DH:]
    else:
        o_ref[...] = o


def _dense(agg, w, b, relu, split_out):
    dout = w.shape[1]
    out_rows = N_CORES * N_PAD if split_out else N_PAD
    out_cols = DH if split_out else dout
    return pl.pallas_call(
        functools.partial(_dense_body, relu=relu, split_out=split_out),
        out_shape=jax.ShapeDtypeStruct((out_rows, out_cols), jnp.float32),
    )(agg, w, b.reshape(1, dout))


def kernel(x, edge_index, W1, b1, W2, b2, W3, b3, W4, b4):
    src = edge_index[0].astype(jnp.int32)
    dst = edge_index[1].astype(jnp.int32)
    pad = E_PAD - N_EDGES
    src_p = jnp.concatenate([src, jnp.zeros((pad,), jnp.int32)])
    dst_p = jnp.concatenate([dst, jnp.full((pad,), N_NODES, jnp.int32)])
    src2 = src_p.reshape(N_SUBCORES, N_CHUNKS, CHUNK)
    dst2 = dst_p.reshape(N_SUBCORES, N_CHUNKS, CHUNK)
    x_pad = jnp.concatenate(
        [x, jnp.zeros((N_PAD - N_NODES, D), jnp.float32)], axis=0)
    f = jnp.concatenate([x_pad[:, :DH], x_pad[:, DH:]], axis=0)
    perm = jnp.asarray(_PERM)
    for w, b, relu, split in ((W1, b1, True, True), (W2, b2, True, True),
                              (W3, b3, True, True), (W4, b4, False, False)):
        fb = f.astype(jnp.bfloat16)[:, perm]
        agg = _sc_agg(f, fb, src2, dst2)
        f = _dense(agg, w, b, relu, split)
    return f[:N_NODES]


# speedup vs baseline: 1.1891x; 1.0497x over previous
import functools

import jax
import jax.numpy as jnp
from jax import lax
from jax.experimental import pallas as pl
from jax.experimental.pallas import tpu as pltpu
from jax.experimental.pallas import tpu_sc as plsc

N_NODES = 10000
D = 128
DH = 64
N_CORES = 2
N_SUBCORES = 16
NW = N_CORES * N_SUBCORES
N_PAD = 10240
RPT = N_PAD // N_SUBCORES
N_EDGES = 320000
CHUNK = 128
EPT = 20480
N_CHUNKS = EPT // CHUNK
E_PAD = N_SUBCORES * EPT
NBUF = 5
NIDX = 2 * NBUF
N_OUTER = N_CHUNKS // NBUF

_mesh = plsc.VectorSubcoreMesh(core_axis_name="c", subcore_axis_name="s")


@functools.partial(
    pl.kernel,
    out_type=jax.ShapeDtypeStruct((N_CORES * N_PAD, DH), jnp.float32),
    mesh=_mesh,
    scratch_types=[
        pltpu.VMEM((NIDX, CHUNK), jnp.int32),
        pltpu.VMEM((NIDX, CHUNK), jnp.int32),
        pltpu.VMEM((NBUF, CHUNK, DH), jnp.float32),
        pltpu.VMEM_SHARED((N_PAD, DH), jnp.float32),
        pltpu.VMEM_SHARED((N_PAD, DH), jnp.float32),
        pltpu.SemaphoreType.DMA((NIDX,)),
        pltpu.SemaphoreType.DMA((NBUF,)),
        pltpu.SemaphoreType.DMA((NBUF,)),
    ],
    compiler_params=pltpu.CompilerParams(use_tc_tiling_on_sc=False,
                                         needs_layout_passes=False),
)
def _sc_agg(f_hbm, src_hbm, dst_hbm, out_hbm,
            src_v, dst_v, rows, feat, acc, isem, gsem, ssem):
    c = lax.axis_index("c")
    s = lax.axis_index("s")

    pltpu.sync_copy(f_hbm.at[pl.ds(c * N_PAD + s * RPT, RPT)],
                    feat.at[pl.ds(s * RPT, RPT)])
    pltpu.sync_copy(f_hbm.at[pl.ds(c * N_PAD + s * RPT, RPT)],
                    acc.at[pl.ds(s * RPT, RPT)])
    plsc.subcore_barrier()

    def islot(i):
        if isinstance(i, int):
            return i % NIDX
        return lax.rem(i, NIDX)

    def idx_start(i):
        j = islot(i)
        pltpu.async_copy(src_hbm.at[s, i], src_v.at[j], isem.at[j])
        pltpu.async_copy(dst_hbm.at[s, i], dst_v.at[j], isem.at[j])

    def idx_wait(i):
        j = islot(i)
        pltpu.make_async_copy(src_hbm.at[s, i], src_v.at[j],
                              isem.at[j]).wait()
        pltpu.make_async_copy(dst_hbm.at[s, i], dst_v.at[j],
                              isem.at[j]).wait()

    def gather_start(i, b):
        pltpu.async_copy(feat.at[src_v.at[islot(i)]], rows.at[b],
                         gsem.at[b])

    def gather_wait(i, b):
        pltpu.make_async_copy(feat.at[src_v.at[islot(i)]], rows.at[b],
                              gsem.at[b]).wait()

    def scatter_start(i, b):
        pltpu.async_copy(rows.at[b], acc.at[dst_v.at[islot(i)]],
                         ssem.at[b], add=True)

    def scatter_wait(i, b):
        pltpu.make_async_copy(rows.at[b], acc.at[dst_v.at[islot(i)]],
                              ssem.at[b]).wait()

    for j in range(NIDX):
        idx_start(j)
    for b in range(NBUF):
        idx_wait(b)
        gather_start(b, b)

    def outer(g, carry):
        i0 = g * NBUF
        for b in range(NBUF):
            gather_wait(i0 + b, b)
            scatter_start(i0 + b, b)
        for b in range(NBUF):
            i2 = i0 + b + NBUF

            @pl.when(i2 < N_CHUNKS)
            def _():
                scatter_wait(i0 + b, b)
                idx_wait(i2)
                gather_start(i2, b)

                @pl.when(i2 + NBUF < N_CHUNKS)
                def _():
                    idx_start(i2 + NBUF)

        return carry

    lax.fori_loop(0, N_OUTER, outer, 0)
    for b in range(NBUF):
        scatter_wait(N_CHUNKS - NBUF + b, b)
    plsc.subcore_barrier()

    pltpu.sync_copy(acc.at[pl.ds(s * RPT, RPT)],
                    out_hbm.at[pl.ds(c * N_PAD + s * RPT, RPT)])


def _dense_body(agg_ref, w_ref, b_ref, o_ref, *, relu, split_out):
    h = jnp.concatenate([agg_ref[0:N_PAD, :], agg_ref[N_PAD:, :]], axis=1)
    o = jnp.dot(h, w_ref[...], preferred_element_type=jnp.float32) + b_ref[...]
    if relu:
        o = jnp.maximum(o, 0.0)
    if split_out:
        o_ref[0:N_PAD, :] = o[:, :DH]
        o_ref[N_PAD:, :] = o[:, DH:]
    else:
        o_ref[...] = o


def _dense(agg, w, b, relu, split_out):
    dout = w.shape[1]
    out_rows = N_CORES * N_PAD if split_out else N_PAD
    out_cols = DH if split_out else dout
    return pl.pallas_call(
        functools.partial(_dense_body, relu=relu, split_out=split_out),
        out_shape=jax.ShapeDtypeStruct((out_rows, out_cols), jnp.float32),
    )(agg, w, b.reshape(1, dout))


def kernel(x, edge_index, W1, b1, W2, b2, W3, b3, W4, b4):
    src = edge_index[0].astype(jnp.int32)
    dst = edge_index[1].astype(jnp.int32)
    pad = E_PAD - N_EDGES
    src_p = jnp.concatenate([src, jnp.zeros((pad,), jnp.int32)])
    dst_p = jnp.concatenate([dst, jnp.full((pad,), N_NODES, jnp.int32)])
    src2 = src_p.reshape(N_SUBCORES, N_CHUNKS, CHUNK)
    dst2 = dst_p.reshape(N_SUBCORES, N_CHUNKS, CHUNK)
    x_pad = jnp.concatenate(
        [x, jnp.zeros((N_PAD - N_NODES, D), jnp.float32)], axis=0)
    f = jnp.concatenate([x_pad[:, :DH], x_pad[:, DH:]], axis=0)
    for w, b, relu, split in ((W1, b1, True, True), (W2, b2, True, True),
                              (W3, b3, True, True), (W4, b4, False, False)):
        agg = _sc_agg(f, src2, dst2)
        f = _dense(agg, w, b, relu, split)
    return f[:N_NODES]
